# baseline (speedup 1.0000x reference)
import functools
import math

import jax
import jax.numpy as jnp
from jax import lax
from jax.experimental import pallas as pl
from jax.experimental.pallas import tpu as pltpu
from jax.experimental.pallas import tpu_sc as plsc

_MODEL_DIM = 64
_BATCH = 16384
_SCALE = math.sqrt(_MODEL_DIM)

_info = plsc.get_sparse_core_info()
_NC = _info.num_cores
_NS = _info.num_subcores
_L = _info.num_lanes
_NW = _NC * _NS
_B_PER_W = _BATCH // _NW
_K = 16

_mesh = plsc.VectorSubcoreMesh(core_axis_name="c", subcore_axis_name="s")


@functools.partial(
    pl.kernel,
    mesh=_mesh,
    out_type=jax.ShapeDtypeStruct((_BATCH, _MODEL_DIM), jnp.float32),
    scratch_types=[
        pltpu.VMEM((_B_PER_W,), jnp.int32),
        pltpu.VMEM((_B_PER_W, _MODEL_DIM), jnp.float32),
        pltpu.SemaphoreType.DMA,
    ],
)
def _emb_lookup(x_hbm, w_hbm, out_hbm, idx_v, rows_v, sem):
    wid = lax.axis_index("s") * _NC + lax.axis_index("c")
    base = wid * _B_PER_W
    pltpu.sync_copy(x_hbm.at[pl.ds(base, _B_PER_W)], idx_v)

    def fetch_rows(i16, carry):
        tv = idx_v[pl.ds(i16 * _K, _K)]
        copies = []
        for j in range(_K):
            i = i16 * _K + j
            copies.append(pltpu.async_copy(w_hbm.at[tv[j]], rows_v.at[i], sem))
        for c in copies:
            c.wait()
        return carry

    lax.fori_loop(0, _B_PER_W // _K, fetch_rows, 0)

    def scale_row(i, carry):
        for j in range(_MODEL_DIM // _L):
            rows_v[i, pl.ds(j * _L, _L)] = rows_v[i, pl.ds(j * _L, _L)] * _SCALE
        return carry

    lax.fori_loop(0, _B_PER_W, scale_row, 0)
    pltpu.sync_copy(rows_v, out_hbm.at[pl.ds(base, _B_PER_W)])


def kernel(x, W):
    return _emb_lookup(x, W)


# speedup vs baseline: 1.4128x; 1.4128x over previous
import functools
import math

import jax
import jax.numpy as jnp
from jax import lax
from jax.experimental import pallas as pl
from jax.experimental.pallas import tpu as pltpu
from jax.experimental.pallas import tpu_sc as plsc

_MODEL_DIM = 64
_BATCH = 16384
_SCALE = math.sqrt(_MODEL_DIM)

_info = plsc.get_sparse_core_info()
_NC = _info.num_cores
_NS = _info.num_subcores
_L = _info.num_lanes
_NW = _NC * _NS
_B_PER_W = _BATCH // _NW
_K = 8
_TCOL = 128

_mesh = plsc.VectorSubcoreMesh(core_axis_name="c", subcore_axis_name="s")


@functools.partial(
    pl.kernel,
    mesh=_mesh,
    compiler_params=pltpu.CompilerParams(needs_layout_passes=False),
    out_type=jax.ShapeDtypeStruct((_MODEL_DIM, _BATCH), jnp.float32),
    scratch_types=[
        pltpu.VMEM((_B_PER_W + _L,), jnp.int32),
        pltpu.VMEM((_K, _MODEL_DIM, _TCOL), jnp.float32),
        pltpu.VMEM((_MODEL_DIM, _B_PER_W), jnp.float32),
        pltpu.SemaphoreType.DMA,
    ],
)
def _emb_lookup(x_hbm, wt_hbm, out_hbm, idx_v, tb, cols_v, sem):
    wid = lax.axis_index("s") * _NC + lax.axis_index("c")
    base = wid * _B_PER_W
    pltpu.sync_copy(x_hbm.at[pl.ds(base, _B_PER_W)], idx_v.at[pl.ds(0, _B_PER_W)])

    f_iota = lax.iota(jnp.int32, _L)

    def round_k(k, carry):
        tv = idx_v[pl.ds(k * _K, _L)]
        copies = []
        for j in range(_K):
            c = jnp.right_shift(tv[j], 7)
            copies.append(
                pltpu.async_copy(
                    wt_hbm.at[:, pl.ds(c * _TCOL, _TCOL)], tb.at[j], sem
                )
            )
        for cp in copies:
            cp.wait()
        for j in range(_K):
            m = jnp.full((_L,), jnp.bitwise_and(tv[j], _TCOL - 1), jnp.int32)
            jv = jnp.full((_L,), j, jnp.int32)
            col = jnp.full((_L,), k * _K + j, jnp.int32)
            for f16 in range(_MODEL_DIM // _L):
                fvec = f_iota + f16 * _L
                vals = plsc.load_gather(tb, [jv, fvec, m]) * _SCALE
                plsc.store_scatter(cols_v, [fvec, col], vals)
        return carry

    lax.fori_loop(0, _B_PER_W // _K, round_k, 0)
    pltpu.sync_copy(cols_v, out_hbm.at[:, pl.ds(base, _B_PER_W)])


def kernel(x, W):
    out_t = _emb_lookup(x, W.T)
    return out_t.T
